# baseline (speedup 1.0000x reference)
import functools

import jax
import jax.numpy as jnp
from jax import lax
from jax.experimental import pallas as pl
from jax.experimental.pallas import tpu as pltpu
from jax.experimental.pallas import tpu_sc as plsc

B = 16384
NUM_ASPECTS = 5
H1 = 128
_TPAD = 1000448

_NC = 2
_NS = 16
_NW = _NC * _NS
_BPW = B // _NW
_CHUNK = 128
_NCHUNK = _BPW // _CHUNK

_BLK = 2048
_OROWS = _BLK // H1


def _make_sc_gather():
    mesh = plsc.VectorSubcoreMesh(core_axis_name="c", subcore_axis_name="s")

    @functools.partial(
        pl.kernel,
        mesh=mesh,
        out_type=[
            jax.ShapeDtypeStruct((B // _CHUNK, _CHUNK), jnp.float32),
            jax.ShapeDtypeStruct((B // _CHUNK, _CHUNK), jnp.float32),
        ],
        scratch_types=[
            pltpu.VMEM((_NCHUNK, _CHUNK), jnp.int32),
            pltpu.VMEM((_NCHUNK, _CHUNK), jnp.int32),
            pltpu.VMEM((_NCHUNK, _CHUNK), jnp.float32),
            pltpu.VMEM((_NCHUNK, _CHUNK), jnp.float32),
            pltpu.SemaphoreType.DMA,
            pltpu.SemaphoreType.DMA,
        ],
    )
    def sc_gather(tabs, uid, iid, uoff_hbm, ioff_hbm, uidx_v, iidx_v,
                  urow_v, irow_v, sem_u, sem_i):
        wid = lax.axis_index("s") * _NC + lax.axis_index("c")
        row0 = wid * _NCHUNK
        pltpu.sync_copy(uid.at[pl.ds(row0, _NCHUNK)], uidx_v)
        pltpu.sync_copy(iid.at[pl.ds(row0, _NCHUNK)], iidx_v)
        for j in range(_NCHUNK):
            for c in range(_CHUNK // 16):
                sl = pl.ds(c * 16, 16)
                iidx_v[j, sl] = iidx_v[j, sl] + _TPAD
        for j in range(_NCHUNK):
            pltpu.async_copy(tabs.at[uidx_v.at[j]], urow_v.at[j], sem_u)
            pltpu.async_copy(tabs.at[iidx_v.at[j]], irow_v.at[j], sem_i)
        for j in range(_NCHUNK):
            pltpu.make_async_copy(tabs.at[uidx_v.at[j]], urow_v.at[j],
                                  sem_u).wait()
            pltpu.make_async_copy(tabs.at[iidx_v.at[j]], irow_v.at[j],
                                  sem_i).wait()
        pltpu.sync_copy(urow_v, uoff_hbm.at[pl.ds(row0, _NCHUNK)])
        pltpu.sync_copy(irow_v, ioff_hbm.at[pl.ds(row0, _NCHUNK)])

    return sc_gather


def _dense_body(u_ref, i_ref, uw_ref, iw_ref, out_ref):
    wt = jnp.transpose(uw_ref[...] * iw_ref[...])
    acc = (u_ref[0] * i_ref[0]) * wt[:, 0:1]
    for k in range(1, NUM_ASPECTS):
        acc = acc + (u_ref[k] * i_ref[k]) * wt[:, k:k + 1]
    r = jnp.sum(acc, axis=1, keepdims=True)
    out_ref[...] = r.reshape(_OROWS, H1)


def _dense_call(u3, i3, uw3, iw3):
    grid = (B // _BLK,)
    return pl.pallas_call(
        _dense_body,
        grid=grid,
        compiler_params=pltpu.CompilerParams(
            vmem_limit_bytes=100 * 1024 * 1024),
        in_specs=[
            pl.BlockSpec((NUM_ASPECTS, _BLK, H1), lambda b: (0, b, 0)),
            pl.BlockSpec((NUM_ASPECTS, _BLK, H1), lambda b: (0, b, 0)),
            pl.BlockSpec((NUM_ASPECTS, _BLK), lambda b: (0, b)),
            pl.BlockSpec((NUM_ASPECTS, _BLK), lambda b: (0, b)),
        ],
        out_specs=pl.BlockSpec((_OROWS, H1), lambda b: (b, 0)),
        out_shape=jax.ShapeDtypeStruct((B // H1, H1), jnp.float32),
    )(u3, i3, uw3, iw3)


def _add_body(r_ref, offu_ref, offi_ref, g_ref, out_ref):
    out_ref[...] = (r_ref[...] + offu_ref[...] + offi_ref[...]
                    + g_ref[0, 0])


def _add_call(r2d, offu, offi, g):
    return pl.pallas_call(
        _add_body,
        in_specs=[
            pl.BlockSpec((B // H1, H1), lambda: (0, 0)),
            pl.BlockSpec((B // H1, H1), lambda: (0, 0)),
            pl.BlockSpec((B // H1, H1), lambda: (0, 0)),
            pl.BlockSpec(memory_space=pltpu.SMEM),
        ],
        out_specs=pl.BlockSpec((B // H1, H1), lambda: (0, 0)),
        out_shape=jax.ShapeDtypeStruct((B // H1, H1), jnp.float32),
    )(r2d, offu, offi, g)


def kernel(userAspRep, itemAspRep, userAspImpt, itemAspImpt, batch_uid,
           batch_iid, user_offset_table, item_offset_table, global_offset):
    npad = _TPAD - user_offset_table.shape[0]
    z = jnp.zeros((1, npad), jnp.float32)
    tabs = jnp.concatenate(
        [jnp.transpose(user_offset_table), z,
         jnp.transpose(item_offset_table), z], axis=1)
    tabs = tabs.reshape(2 * _TPAD)
    uid = jnp.asarray(batch_uid, jnp.int32).reshape(B // _CHUNK, _CHUNK)
    iid = jnp.asarray(batch_iid, jnp.int32).reshape(B // _CHUNK, _CHUNK)

    uoff, ioff = _make_sc_gather()(tabs, uid, iid)

    u3 = jnp.transpose(userAspRep, (1, 0, 2))
    i3 = jnp.transpose(itemAspRep, (1, 0, 2))
    uw3 = jnp.transpose(userAspImpt)
    iw3 = jnp.transpose(itemAspImpt)
    g = global_offset.reshape(1, 1)

    r2d = _dense_call(u3, i3, uw3, iw3)
    out = _add_call(r2d, uoff, ioff, g)
    return out.reshape(B, 1)


# speedup vs baseline: 2.9347x; 2.9347x over previous
import functools

import jax
import jax.numpy as jnp
from jax import lax
from jax.experimental import pallas as pl
from jax.experimental.pallas import tpu as pltpu
from jax.experimental.pallas import tpu_sc as plsc

B = 16384
NUM_ASPECTS = 5
H1 = 128
_TPAD = 1000448

_NC = 2
_NS = 16
_NW = _NC * _NS
_BPW = B // _NW
_CHUNK = 128
_NCHUNK = _BPW // _CHUNK

_BLK = 2048
_OROWS = _BLK // H1


def _make_sc_gather():
    mesh = plsc.VectorSubcoreMesh(core_axis_name="c", subcore_axis_name="s")

    @functools.partial(
        pl.kernel,
        mesh=mesh,
        out_type=[
            jax.ShapeDtypeStruct((B // _CHUNK, _CHUNK), jnp.float32),
            jax.ShapeDtypeStruct((B // _CHUNK, _CHUNK), jnp.float32),
        ],
        scratch_types=[
            pltpu.VMEM((_NCHUNK, _CHUNK), jnp.int32),
            pltpu.VMEM((_NCHUNK, _CHUNK), jnp.int32),
            pltpu.VMEM((_NCHUNK, _CHUNK), jnp.float32),
            pltpu.VMEM((_NCHUNK, _CHUNK), jnp.float32),
            pltpu.SemaphoreType.DMA,
            pltpu.SemaphoreType.DMA,
        ],
    )
    def sc_gather(utab, itab, uid, iid, uoff_hbm, ioff_hbm, uidx_v, iidx_v,
                  urow_v, irow_v, sem_u, sem_i):
        wid = lax.axis_index("s") * _NC + lax.axis_index("c")
        row0 = wid * _NCHUNK
        pltpu.sync_copy(uid.at[pl.ds(row0, _NCHUNK)], uidx_v)
        pltpu.sync_copy(iid.at[pl.ds(row0, _NCHUNK)], iidx_v)
        for j in range(_NCHUNK):
            pltpu.async_copy(utab.at[uidx_v.at[j]], urow_v.at[j], sem_u)
            pltpu.async_copy(itab.at[iidx_v.at[j]], irow_v.at[j], sem_i)
        for j in range(_NCHUNK):
            pltpu.make_async_copy(utab.at[uidx_v.at[j]], urow_v.at[j],
                                  sem_u).wait()
            pltpu.make_async_copy(itab.at[iidx_v.at[j]], irow_v.at[j],
                                  sem_i).wait()
        pltpu.sync_copy(urow_v, uoff_hbm.at[pl.ds(row0, _NCHUNK)])
        pltpu.sync_copy(irow_v, ioff_hbm.at[pl.ds(row0, _NCHUNK)])

    return sc_gather


def _dense_body(u_ref, i_ref, uw_ref, iw_ref, out_ref):
    wt = jnp.transpose(uw_ref[...] * iw_ref[...])
    acc = (u_ref[0] * i_ref[0]) * wt[:, 0:1]
    for k in range(1, NUM_ASPECTS):
        acc = acc + (u_ref[k] * i_ref[k]) * wt[:, k:k + 1]
    r = jnp.sum(acc, axis=1, keepdims=True)
    out_ref[...] = r.reshape(_OROWS, H1)


def _dense_call(u3, i3, uw3, iw3):
    grid = (B // _BLK,)
    return pl.pallas_call(
        _dense_body,
        grid=grid,
        compiler_params=pltpu.CompilerParams(
            vmem_limit_bytes=100 * 1024 * 1024),
        in_specs=[
            pl.BlockSpec((NUM_ASPECTS, _BLK, H1), lambda b: (0, b, 0)),
            pl.BlockSpec((NUM_ASPECTS, _BLK, H1), lambda b: (0, b, 0)),
            pl.BlockSpec((NUM_ASPECTS, _BLK), lambda b: (0, b)),
            pl.BlockSpec((NUM_ASPECTS, _BLK), lambda b: (0, b)),
        ],
        out_specs=pl.BlockSpec((_OROWS, H1), lambda b: (b, 0)),
        out_shape=jax.ShapeDtypeStruct((B // H1, H1), jnp.float32),
    )(u3, i3, uw3, iw3)


def _add_body(r_ref, offu_ref, offi_ref, g_ref, out_ref):
    out_ref[...] = (r_ref[...] + offu_ref[...] + offi_ref[...]
                    + g_ref[0, 0])


def _add_call(r2d, offu, offi, g):
    return pl.pallas_call(
        _add_body,
        in_specs=[
            pl.BlockSpec((B // H1, H1), lambda: (0, 0)),
            pl.BlockSpec((B // H1, H1), lambda: (0, 0)),
            pl.BlockSpec((B // H1, H1), lambda: (0, 0)),
            pl.BlockSpec(memory_space=pltpu.SMEM),
        ],
        out_specs=pl.BlockSpec((B // H1, H1), lambda: (0, 0)),
        out_shape=jax.ShapeDtypeStruct((B // H1, H1), jnp.float32),
    )(r2d, offu, offi, g)


def kernel(userAspRep, itemAspRep, userAspImpt, itemAspImpt, batch_uid,
           batch_iid, user_offset_table, item_offset_table, global_offset):
    npad = _TPAD - user_offset_table.shape[0]
    utab = jnp.pad(jnp.transpose(user_offset_table), ((0, 0), (0, npad)))
    itab = jnp.pad(jnp.transpose(item_offset_table), ((0, 0), (0, npad)))
    utab = utab.reshape(_TPAD)
    itab = itab.reshape(_TPAD)
    uid = jnp.asarray(batch_uid, jnp.int32).reshape(B // _CHUNK, _CHUNK)
    iid = jnp.asarray(batch_iid, jnp.int32).reshape(B // _CHUNK, _CHUNK)

    uoff, ioff = _make_sc_gather()(utab, itab, uid, iid)

    u3 = jnp.transpose(userAspRep, (1, 0, 2))
    i3 = jnp.transpose(itemAspRep, (1, 0, 2))
    uw3 = jnp.transpose(userAspImpt)
    iw3 = jnp.transpose(itemAspImpt)
    g = global_offset.reshape(1, 1)

    r2d = _dense_call(u3, i3, uw3, iw3)
    out = _add_call(r2d, uoff, ioff, g)
    return out.reshape(B, 1)
